# speedup vs baseline: 1.9830x; 1.9830x over previous
import jax
import jax.numpy as jnp
from jax.experimental import pallas as pl

_CB = 8


def _body(x_ref, o_ref):
    i = pl.program_id(0)
    x = x_ref[...]
    cb, hh, ww = x.shape
    c_local = jax.lax.broadcasted_iota(jnp.int32, x.shape, 0)
    row = jax.lax.broadcasted_iota(jnp.int32, x.shape, 1)
    col = jax.lax.broadcasted_iota(jnp.int32, x.shape, 2)
    c = (i * cb + c_local) % 96
    cond = (c < 64) & (row == (7 * c) % hh) & (col == (13 * c) % ww)
    o_ref[...] = jnp.where(cond, 0.0, x)


def kernel(X):
    B, C, H, W = X.shape
    Xf = X.reshape(B * C, H, W)
    out = pl.pallas_call(
        _body,
        grid=(B * C // _CB,),
        in_specs=[pl.BlockSpec((_CB, H, W), lambda i: (i, 0, 0))],
        out_specs=pl.BlockSpec((_CB, H, W), lambda i: (i, 0, 0)),
        out_shape=jax.ShapeDtypeStruct((B * C, H, W), X.dtype),
    )(Xf)
    return out.reshape(B, C, H, W)
